# baseline (speedup 1.0000x reference)
import functools

import jax
import jax.numpy as jnp
from jax import lax
from jax.experimental import pallas as pl
from jax.experimental.pallas import tpu as pltpu
from jax.experimental.pallas import tpu_sc as plsc

N_ROIS = 268
D_MODEL = 64
BATCH = 4096
ROWS = N_ROIS * D_MODEL
NW = 32
CHUNK = 8
NCHUNK = ROWS // CHUNK
CPW = NCHUNK // NW


def _chunk_dst(out_hbm, c):
    r = lax.div(c, D_MODEL // CHUNK)
    d0 = lax.rem(c, D_MODEL // CHUNK) * CHUNK
    return out_hbm.at[r, pl.ds(d0, CHUNK)]


def _sc_body(tab_hbm, out_hbm, tab_v, bufs, sems):
    wid = lax.axis_index("c") * 16 + lax.axis_index("s")
    pltpu.sync_copy(tab_hbm, tab_v.at[pl.ds(0, ROWS)])

    def chunk_body(t, carry):
        c = wid * CPW + t
        slot = lax.rem(t, 2)

        @pl.when(t >= 2)
        def _():
            pltpu.make_async_copy(
                bufs.at[slot], _chunk_dst(out_hbm, c - 2), sems.at[slot]
            ).wait()

        vec = tab_v[pl.ds(c * CHUNK, 16)]
        bufs[slot, 0, pl.ds(0, 16)] = vec

        pltpu.make_async_copy(
            bufs.at[slot], _chunk_dst(out_hbm, c), sems.at[slot]
        ).start()
        return carry

    lax.fori_loop(0, CPW, chunk_body, 0)

    for tail in (CPW - 2, CPW - 1):
        c = wid * CPW + tail
        pltpu.make_async_copy(
            bufs.at[tail % 2], _chunk_dst(out_hbm, c), sems.at[tail % 2]
        ).wait()


def kernel(batch_size, pos_embedding):
    mesh = plsc.VectorSubcoreMesh(core_axis_name="c", subcore_axis_name="s")
    run = functools.partial(
        pl.kernel,
        mesh=mesh,
        out_type=jax.ShapeDtypeStruct((N_ROIS, D_MODEL, BATCH), jnp.float32),
        scratch_types=[
            pltpu.VMEM((ROWS + 16,), jnp.float32),
            pltpu.VMEM((2, CHUNK, BATCH), jnp.float32),
            pltpu.SemaphoreType.DMA((2,)),
        ],
    )(_sc_body)
    out = run(pos_embedding.reshape(ROWS))
    return jnp.transpose(out, (2, 0, 1))


# speedup vs baseline: 1.0235x; 1.0235x over previous
import functools

import jax
import jax.numpy as jnp
from jax import lax
from jax.experimental import pallas as pl
from jax.experimental.pallas import tpu as pltpu
from jax.experimental.pallas import tpu_sc as plsc

N_ROIS = 268
D_MODEL = 64
BATCH = 4096
ROWS = N_ROIS * D_MODEL
NW = 32
CHUNK = 8
NCHUNK = ROWS // CHUNK
CPW = NCHUNK // NW


def _chunk_dst(out_hbm, c):
    r = lax.div(c, D_MODEL // CHUNK)
    d0 = lax.rem(c, D_MODEL // CHUNK) * CHUNK
    return out_hbm.at[r, pl.ds(d0, CHUNK)]


def _sc_body(tab_hbm, out_hbm, tab_v, bufs, sems):
    wid = lax.axis_index("c") * 16 + lax.axis_index("s")
    pltpu.sync_copy(tab_hbm, tab_v.at[pl.ds(0, ROWS)])

    def chunk_body(t, carry):
        c = wid * CPW + t
        slot = lax.rem(t, 2)

        @pl.when(t >= 2)
        def _():
            pltpu.make_async_copy(
                bufs.at[slot], _chunk_dst(out_hbm, c - 2), sems.at[slot]
            ).wait()

        vec = tab_v[pl.ds(c * CHUNK, 16)]

        def fill(j, cc):
            for u in range(CHUNK):
                vsplat = jnp.full((16,), vec[u], jnp.float32)
                for jj in range(16):
                    bufs[slot, u, pl.ds(j * 256 + jj * 16, 16)] = vsplat
            return cc

        lax.fori_loop(0, BATCH // 256, fill, 0)

        pltpu.make_async_copy(
            bufs.at[slot], _chunk_dst(out_hbm, c), sems.at[slot]
        ).start()
        return carry

    lax.fori_loop(0, CPW, chunk_body, 0)

    for tail in (CPW - 2, CPW - 1):
        c = wid * CPW + tail
        pltpu.make_async_copy(
            bufs.at[tail % 2], _chunk_dst(out_hbm, c), sems.at[tail % 2]
        ).wait()


def kernel(batch_size, pos_embedding):
    mesh = plsc.VectorSubcoreMesh(core_axis_name="c", subcore_axis_name="s")
    run = functools.partial(
        pl.kernel,
        mesh=mesh,
        out_type=jax.ShapeDtypeStruct((N_ROIS, D_MODEL, BATCH), jnp.float32),
        scratch_types=[
            pltpu.VMEM((ROWS + 16,), jnp.float32),
            pltpu.VMEM((2, CHUNK, BATCH), jnp.float32),
            pltpu.SemaphoreType.DMA((2,)),
        ],
    )(_sc_body)
    out = run(pos_embedding.reshape(ROWS))
    return jnp.transpose(out, (2, 0, 1))
